# Initial kernel scaffold; baseline (speedup 1.0000x reference)
import jax
import jax.numpy as jnp
from jax.experimental import pallas as pl


def kernel(inputs, W0, W1, V):
    raise NotImplementedError("write your pallas kernel here")


# speedup vs baseline: 2.2649x; 2.2649x over previous
import functools

import jax
import jax.numpy as jnp
from jax import lax
from jax.experimental import pallas as pl
from jax.experimental.pallas import tpu as pltpu
from jax.experimental.pallas import tpu_sc as plsc

B = 16384
F = 26
K = 32
NC = 2
NS = 16
NW = NC * NS
BPW = B // NW
RPC = 4
IPC = RPC * F
NCHUNK = BPW // RPC
NBUF = 4
W1PAD = 112


def _fm_body(idx_hbm, w0_hbm, w1_hbm, v_hbm, out_hbm,
             idx_v, vrows, w1rows, outv, w0v, *sems):
    sem_v = sems[:NBUF]
    sem_w = sems[NBUF:]
    wid = lax.axis_index("s") * NC + lax.axis_index("c")

    pltpu.sync_copy(idx_hbm.at[wid], idx_v)
    pltpu.sync_copy(w0_hbm, w0v)

    zero16 = jnp.zeros((16,), jnp.float32)
    for b in range(NBUF):
        w1rows[b, pl.ds(96, 16)] = zero16

    lane = lax.iota(jnp.int32, 16)
    m10 = jnp.where(lane < 10, 1.0, 0.0).astype(jnp.float32)
    w0s = w0v[pl.ds(0, 16)][0]
    out_mask = lane < RPC
    lane_mod = lane & (RPC - 1)

    def v_copy(g, b):
        return pltpu.make_async_copy(v_hbm.at[idx_v.at[g]], vrows.at[b], sem_v[b])

    def w_copy(g, b):
        return pltpu.make_async_copy(
            w1_hbm.at[idx_v.at[g]], w1rows.at[b, pl.ds(0, IPC)], sem_w[b])

    for b in range(NBUF):
        v_copy(b, b).start()
        w_copy(b, b).start()

    def chunk_body(i, carry):
        g0 = i * NBUF
        for b in range(NBUF):
            g = g0 + b
            v_copy(g, b).wait()
            w_copy(g, b).wait()
            vals = zero16
            for r in range(RPC):
                o = r * F
                x0 = vrows[b, o, pl.ds(0, 16)]
                x1 = vrows[b, o, pl.ds(16, 16)]
                s0, s1 = x0, x1
                q0, q1 = x0 * x0, x1 * x1
                for f in range(1, F):
                    x0 = vrows[b, o + f, pl.ds(0, 16)]
                    x1 = vrows[b, o + f, pl.ds(16, 16)]
                    s0 += x0
                    s1 += x1
                    q0 += x0 * x0
                    q1 += x1 * x1
                t = s0 * s0 + s1 * s1 - q0 - q1
                la = w1rows[b, pl.ds(F * r, 16)]
                lb = w1rows[b, pl.ds(F * r + 16, 16)] * m10
                val = jnp.sum(0.5 * t + la + lb) + w0s
                vals = jnp.where(lane == r, val, vals)
            plsc.store_scatter(outv, [g * RPC + lane_mod], vals, mask=out_mask)
            nxt = g + NBUF

            @pl.when(nxt < NCHUNK)
            def _():
                v_copy(nxt, b).start()
                w_copy(nxt, b).start()
        return carry

    lax.fori_loop(0, NCHUNK // NBUF, chunk_body, 0)
    pltpu.sync_copy(outv, out_hbm.at[pl.ds(wid * BPW, BPW)])


@jax.jit
def _fm(idx, w0b, w1f, V):
    mesh = plsc.VectorSubcoreMesh(core_axis_name="c", subcore_axis_name="s")
    run = functools.partial(
        pl.kernel,
        out_type=jax.ShapeDtypeStruct((B,), jnp.float32),
        mesh=mesh,
        scratch_types=[
            pltpu.VMEM((NCHUNK, IPC), jnp.int32),
            pltpu.VMEM((NBUF, IPC, K), jnp.float32),
            pltpu.VMEM((NBUF, W1PAD), jnp.float32),
            pltpu.VMEM((BPW,), jnp.float32),
            pltpu.VMEM((16,), jnp.float32),
        ] + [pltpu.SemaphoreType.DMA] * (2 * NBUF),
        compiler_params=pltpu.CompilerParams(
            needs_layout_passes=False, use_tc_tiling_on_sc=False),
    )(_fm_body)
    return run(idx, w0b, w1f, V)


def kernel(inputs, W0, W1, V):
    idx = inputs.astype(jnp.int32).reshape(NW, NCHUNK, IPC)
    w0b = jnp.broadcast_to(W0.astype(jnp.float32), (16,))
    w1f = W1.reshape(-1)
    out = _fm(idx, w0b, w1f, V)
    return out.reshape(B, 1)
